# baseline (device time: 6210 ns/iter reference)
import jax
import jax.numpy as jnp
from jax import lax
from jax.experimental import pallas as pl
from jax.experimental.pallas import tpu as pltpu

N_DEV = 16


def kernel(x):
    m, n = x.shape

    def body(x_ref, out_ref, halo_up, halo_dn, send_sems, recv_sems):
        p = lax.axis_index("i")
        has_left = p > 0
        has_right = p < N_DEV - 1

        barrier = pltpu.get_barrier_semaphore()

        @pl.when(has_left)
        def _():
            pl.semaphore_signal(
                barrier, inc=1,
                device_id=(p - 1,), device_id_type=pl.DeviceIdType.MESH,
            )

        @pl.when(jnp.logical_not(has_left))
        def _():
            pl.semaphore_signal(barrier, inc=1)

        @pl.when(has_right)
        def _():
            pl.semaphore_signal(
                barrier, inc=1,
                device_id=(p + 1,), device_id_type=pl.DeviceIdType.MESH,
            )

        @pl.when(jnp.logical_not(has_right))
        def _():
            pl.semaphore_signal(barrier, inc=1)

        pl.semaphore_wait(barrier, 2)

        @pl.when(has_right)
        def _():
            fwd = pltpu.make_async_remote_copy(
                src_ref=x_ref.at[pl.ds(m - 1, 1), :],
                dst_ref=halo_up,
                send_sem=send_sems.at[0],
                recv_sem=recv_sems.at[0],
                device_id=(p + 1,),
                device_id_type=pl.DeviceIdType.MESH,
            )
            fwd.start()

        @pl.when(has_left)
        def _():
            bwd = pltpu.make_async_remote_copy(
                src_ref=x_ref.at[pl.ds(0, 1), :],
                dst_ref=halo_dn,
                send_sem=send_sems.at[1],
                recv_sem=recv_sems.at[1],
                device_id=(p - 1,),
                device_id_type=pl.DeviceIdType.MESH,
            )
            bwd.start()

        @pl.when(jnp.logical_not(has_left))
        def _():
            halo_up[:, :] = 2.0 * x_ref[pl.ds(0, 1), :] - x_ref[pl.ds(1, 1), :]

        @pl.when(jnp.logical_not(has_right))
        def _():
            halo_dn[:, :] = (
                2.0 * x_ref[pl.ds(m - 1, 1), :] - x_ref[pl.ds(m - 2, 1), :]
            )

        out_ref[pl.ds(1, m - 2), :] = x_ref[pl.ds(1, m - 2), :]

        @pl.when(has_left)
        def _():
            recv_up = pltpu.make_async_remote_copy(
                src_ref=x_ref.at[pl.ds(m - 1, 1), :],
                dst_ref=halo_up,
                send_sem=send_sems.at[0],
                recv_sem=recv_sems.at[0],
                device_id=(p,),
                device_id_type=pl.DeviceIdType.MESH,
            )
            recv_up.wait_recv()

        out_ref[pl.ds(0, 1), :] = (
            0.25 * halo_up[:, :]
            + 0.5 * x_ref[pl.ds(0, 1), :]
            + 0.25 * x_ref[pl.ds(1, 1), :]
        )

        @pl.when(has_right)
        def _():
            recv_dn = pltpu.make_async_remote_copy(
                src_ref=x_ref.at[pl.ds(0, 1), :],
                dst_ref=halo_dn,
                send_sem=send_sems.at[1],
                recv_sem=recv_sems.at[1],
                device_id=(p,),
                device_id_type=pl.DeviceIdType.MESH,
            )
            recv_dn.wait_recv()

        out_ref[pl.ds(m - 1, 1), :] = (
            0.25 * x_ref[pl.ds(m - 2, 1), :]
            + 0.5 * x_ref[pl.ds(m - 1, 1), :]
            + 0.25 * halo_dn[:, :]
        )

        @pl.when(has_right)
        def _():
            fwd_done = pltpu.make_async_remote_copy(
                src_ref=x_ref.at[pl.ds(m - 1, 1), :],
                dst_ref=halo_up,
                send_sem=send_sems.at[0],
                recv_sem=recv_sems.at[0],
                device_id=(p + 1,),
                device_id_type=pl.DeviceIdType.MESH,
            )
            fwd_done.wait_send()

        @pl.when(has_left)
        def _():
            bwd_done = pltpu.make_async_remote_copy(
                src_ref=x_ref.at[pl.ds(0, 1), :],
                dst_ref=halo_dn,
                send_sem=send_sems.at[1],
                recv_sem=recv_sems.at[1],
                device_id=(p - 1,),
                device_id_type=pl.DeviceIdType.MESH,
            )
            bwd_done.wait_send()

    return pl.pallas_call(
        body,
        out_shape=jax.ShapeDtypeStruct((m, n), x.dtype),
        in_specs=[pl.BlockSpec(memory_space=pltpu.VMEM)],
        out_specs=pl.BlockSpec(memory_space=pltpu.VMEM),
        scratch_shapes=[
            pltpu.VMEM((1, n), x.dtype),
            pltpu.VMEM((1, n), x.dtype),
            pltpu.SemaphoreType.DMA((2,)),
            pltpu.SemaphoreType.DMA((2,)),
        ],
        compiler_params=pltpu.CompilerParams(collective_id=0),
    )(x)


# device time: 1805 ns/iter; 3.4404x vs baseline; 3.4404x over previous
import jax
import jax.numpy as jnp
from jax import lax
from jax.experimental import pallas as pl
from jax.experimental.pallas import tpu as pltpu

N_DEV = 16


def kernel(x):
    m, n = x.shape

    def body(x_ref, out_ref):
        out_ref[pl.ds(1, m - 2), :] = (
            0.25 * x_ref[pl.ds(0, m - 2), :]
            + 0.5 * x_ref[pl.ds(1, m - 2), :]
            + 0.25 * x_ref[pl.ds(2, m - 2), :]
        )
        out_ref[pl.ds(0, 1), :] = x_ref[pl.ds(0, 1), :]
        out_ref[pl.ds(m - 1, 1), :] = x_ref[pl.ds(m - 1, 1), :]

    return pl.pallas_call(
        body,
        out_shape=jax.ShapeDtypeStruct((m, n), x.dtype),
        in_specs=[pl.BlockSpec(memory_space=pltpu.VMEM)],
        out_specs=pl.BlockSpec(memory_space=pltpu.VMEM),
    )(x)
